# baseline (speedup 1.0000x reference)
import functools

import jax
import jax.numpy as jnp
from jax import lax
from jax.experimental import pallas as pl
from jax.experimental.pallas import tpu as pltpu
from jax.experimental.pallas import tpu_sc as plsc

EMBED_DIM = 32
PAD_DIM = 128
BATCH = 16384
NUM_CORES = 2
NUM_SUBCORES = 16
NUM_WORKERS = NUM_CORES * NUM_SUBCORES
CHUNK = 128
B_PER_W = BATCH // NUM_WORKERS
NCHUNK = B_PER_W // CHUNK
LANES = 16

_mesh = plsc.VectorSubcoreMesh(core_axis_name="c", subcore_axis_name="s")


@functools.partial(
    pl.kernel,
    mesh=_mesh,
    out_type=jax.ShapeDtypeStruct((EMBED_DIM, BATCH), jnp.float32),
    scratch_types=[
        pltpu.VMEM((B_PER_W,), jnp.int32),
        pltpu.VMEM((B_PER_W, PAD_DIM), jnp.float32),
        pltpu.VMEM((EMBED_DIM, B_PER_W), jnp.float32),
        pltpu.SemaphoreType.DMA,
    ],
    compiler_params=pltpu.CompilerParams(needs_layout_passes=False),
)
def _gather_kernel(idx_hbm, table_hbm, out_hbm, idx_v, rows_v, stage_v, sem):
    wid = lax.axis_index("s") * NUM_CORES + lax.axis_index("c")
    base = wid * B_PER_W
    pltpu.sync_copy(idx_hbm.at[pl.ds(base, B_PER_W)], idx_v)
    copies = [
        pltpu.async_copy(
            table_hbm.at[idx_v.at[pl.ds(j * CHUNK, CHUNK)]],
            rows_v.at[pl.ds(j * CHUNK, CHUNK)],
            sem,
        )
        for j in range(NCHUNK)
    ]
    for c in copies:
        c.wait()

    lane_iota = lax.iota(jnp.int32, LANES)

    def body(g, _):
        row0 = g * LANES
        row_idx = lane_iota + row0
        for d in range(EMBED_DIM):
            vals = plsc.load_gather(
                rows_v, [row_idx, jnp.full((LANES,), d, jnp.int32)]
            )
            stage_v[d, pl.ds(row0, LANES)] = vals
        return _

    lax.fori_loop(0, B_PER_W // LANES, body, None)

    pltpu.sync_copy(stage_v, out_hbm.at[:, pl.ds(base, B_PER_W)])


def kernel(indices, table):
    table_pad = jnp.pad(table, ((0, 0), (0, PAD_DIM - EMBED_DIM)))
    out_t = _gather_kernel(indices.astype(jnp.int32), table_pad)
    return out_t.T


# speedup vs baseline: 1.2826x; 1.2826x over previous
import functools

import jax
import jax.numpy as jnp
from jax import lax
from jax.experimental import pallas as pl
from jax.experimental.pallas import tpu as pltpu
from jax.experimental.pallas import tpu_sc as plsc

EMBED_DIM = 32
BATCH = 16384
NUM_CORES = 2
NUM_SUBCORES = 16
NUM_WORKERS = NUM_CORES * NUM_SUBCORES
B_PER_W = BATCH // NUM_WORKERS
LANES = 16

_mesh = plsc.VectorSubcoreMesh(core_axis_name="c", subcore_axis_name="s")


@functools.partial(
    pl.kernel,
    mesh=_mesh,
    out_type=jax.ShapeDtypeStruct((EMBED_DIM, BATCH), jnp.float32),
    scratch_types=[
        pltpu.VMEM((B_PER_W,), jnp.int32),
        pltpu.VMEM((B_PER_W, EMBED_DIM), jnp.float32),
        pltpu.VMEM((EMBED_DIM, B_PER_W), jnp.float32),
        pltpu.SemaphoreType.DMA,
    ],
    compiler_params=pltpu.CompilerParams(needs_layout_passes=False),
)
def _gather_kernel(idx_hbm, table_hbm, out_hbm, idx_v, rows_v, stage_v, sem):
    wid = lax.axis_index("s") * NUM_CORES + lax.axis_index("c")
    base = wid * B_PER_W
    pltpu.sync_copy(idx_hbm.at[pl.ds(base, B_PER_W)], idx_v)

    def fire(g, _):
        p0 = g * LANES
        vec = idx_v[pl.ds(p0, LANES)]
        for k in range(LANES):
            pltpu.async_copy(
                table_hbm.at[pl.ds(vec[k], 1), :],
                rows_v.at[pl.ds(p0 + k, 1), :],
                sem,
            )
        return _

    lax.fori_loop(0, B_PER_W // LANES, fire, None)
    pltpu.make_async_copy(
        table_hbm.at[pl.ds(0, B_PER_W), :], rows_v, sem
    ).wait()

    lane_iota = lax.iota(jnp.int32, LANES)

    def body(g, _):
        row0 = g * LANES
        row_idx = lane_iota + row0
        for d in range(EMBED_DIM):
            vals = plsc.load_gather(
                rows_v, [row_idx, jnp.full((LANES,), d, jnp.int32)]
            )
            stage_v[d, pl.ds(row0, LANES)] = vals
        return _

    lax.fori_loop(0, B_PER_W // LANES, body, None)

    pltpu.sync_copy(stage_v, out_hbm.at[:, pl.ds(base, B_PER_W)])


def kernel(indices, table):
    out_t = _gather_kernel(indices.astype(jnp.int32), table)
    return out_t.T
